# baseline (speedup 1.0000x reference)
import functools

import jax
import jax.numpy as jnp
from jax import lax
from jax.experimental import pallas as pl
from jax.experimental.pallas import tpu as pltpu
from jax.experimental.pallas import tpu_sc as plsc

NC = 2
NS = 16
NW = NC * NS
CH = 128
G = 8
D = 128


def _sc_agg_body(ng, agg_rows,
                 h_hbm, src_hbm, dst_hbm, out_hbm,
                 src_g0, src_g1, dst_g0, dst_g1, msg0, msg1,
                 agg_sh, isem, gsem0, gsem1):
    c = lax.axis_index("c")
    s = lax.axis_index("s")
    wid = c * NS + s

    src_g = (src_g0, src_g1)
    dst_g = (dst_g0, dst_g1)
    msg = (msg0, msg1)
    gsem = (gsem0, gsem1)

    zrows = agg_rows // NS

    def _zero_row(i, _):
        for j in range(D // 16):
            msg0[i, pl.ds(j * 16, 16)] = jnp.zeros((16,), jnp.float32)
        return 0

    lax.fori_loop(0, CH, _zero_row, 0)
    for k in range(zrows // CH):
        pltpu.sync_copy(msg0, agg_sh.at[pl.ds(s * zrows + k * CH, CH)])
    rem = zrows % CH
    if rem:
        pltpu.sync_copy(msg0.at[pl.ds(0, rem)],
                        agg_sh.at[pl.ds(s * zrows + (zrows // CH) * CH, rem)])
    plsc.subcore_barrier()

    row0 = wid * (ng * G)

    def _gather_start(src_buf, q, mi):
        return pltpu.async_copy(h_hbm.at[src_buf.at[q]], msg[mi], gsem[mi])

    def _gather_wait(src_buf, q, mi):
        pltpu.make_async_copy(h_hbm.at[src_buf.at[q]], msg[mi],
                              gsem[mi]).wait()

    pltpu.sync_copy(src_hbm.at[pl.ds(row0, G)], src_g[0])
    pltpu.sync_copy(dst_hbm.at[pl.ds(row0, G)], dst_g[0])
    _gather_start(src_g[0], 0, 0)

    def _pair(p, _):
        for gq in (0, 1):
            g = 2 * p + gq
            sg, dg = src_g[gq], dst_g[gq]
            sg_n, dg_n = src_g[1 - gq], dst_g[1 - gq]
            @pl.when(g < ng - 1)
            def _():
                grow = row0 + (g + 1) * G
                pltpu.async_copy(src_hbm.at[pl.ds(grow, G)], sg_n, isem)
                pltpu.async_copy(dst_hbm.at[pl.ds(grow, G)], dg_n, isem)

            for q in range(G):
                mi = q % 2
                if q < G - 1:
                    _gather_start(sg, q + 1, 1 - mi)
                else:
                    @pl.when(g < ng - 1)
                    def _():
                        pltpu.make_async_copy(
                            src_hbm.at[pl.ds(row0, G)], sg_n, isem).wait()
                        pltpu.make_async_copy(
                            dst_hbm.at[pl.ds(row0, G)], dg_n, isem).wait()
                        _gather_start(sg_n, 0, 1 - mi)
                _gather_wait(sg, q, mi)
                pltpu.sync_copy(msg[mi], agg_sh.at[dg.at[q]], add=True)
        return 0

    lax.fori_loop(0, ng // 2, _pair, 0)
    plsc.subcore_barrier()

    pltpu.sync_copy(agg_sh.at[pl.ds(s * zrows, zrows)],
                    out_hbm.at[c, pl.ds(s * zrows, zrows)])


@functools.partial(jax.jit, static_argnums=(3,))
def _sc_agg(h, src_rows, dst_rows, n_nodes):
    r = src_rows.shape[0]
    ng = (r - G) // (NW * G)
    agg_rows = ((n_nodes + 1 + NS * 8 - 1) // (NS * 8)) * (NS * 8)
    mesh = plsc.VectorSubcoreMesh(core_axis_name="c", subcore_axis_name="s",
                                  num_cores=NC, num_subcores=NS)
    body = functools.partial(_sc_agg_body, ng, agg_rows)
    kern = pl.kernel(
        body,
        out_type=jax.ShapeDtypeStruct((NC, agg_rows, D), jnp.float32),
        mesh=mesh,
        scratch_types=[
            pltpu.VMEM((G, CH), jnp.int32),
            pltpu.VMEM((G, CH), jnp.int32),
            pltpu.VMEM((G, CH), jnp.int32),
            pltpu.VMEM((G, CH), jnp.int32),
            pltpu.VMEM((CH, D), jnp.float32),
            pltpu.VMEM((CH, D), jnp.float32),
            pltpu.VMEM_SHARED((agg_rows, D), jnp.float32),
            pltpu.SemaphoreType.DMA,
            pltpu.SemaphoreType.DMA,
            pltpu.SemaphoreType.DMA,
        ],
    )
    return kern(h, src_rows, dst_rows)


def _tc_body(h_ref, a0_ref, a1_ref, w_ref, o_ref):
    agg = a0_ref[0] + a1_ref[0]
    t = jnp.dot(agg, w_ref[...], preferred_element_type=jnp.float32)
    o_ref[...] = h_ref[...] + jnp.maximum(t, 0.0)


def _tc_update(h, agg2, w):
    n = h.shape[0]
    blk = 1000
    grid = (n // blk,)
    return pl.pallas_call(
        _tc_body,
        grid=grid,
        in_specs=[
            pl.BlockSpec((blk, D), lambda i: (i, 0)),
            pl.BlockSpec((1, blk, D), lambda i: (0, i, 0)),
            pl.BlockSpec((1, blk, D), lambda i: (1, i, 0)),
            pl.BlockSpec((D, D), lambda i: (0, 0)),
        ],
        out_specs=pl.BlockSpec((blk, D), lambda i: (i, 0)),
        out_shape=jax.ShapeDtypeStruct((n, D), jnp.float32),
    )(h, agg2, agg2, w)


def kernel(x, edge_index, W):
    n = x.shape[0]
    e = edge_index.shape[1]
    src = edge_index[0].astype(jnp.int32)
    dst = edge_index[1].astype(jnp.int32)

    gsz = G * CH
    per_w = ((e + NW - 1) // NW + gsz - 1) // gsz * gsz
    e_pad = per_w * NW + gsz
    src_pad = jnp.concatenate([src, jnp.zeros((e_pad - e,), jnp.int32)])
    dst_pad = jnp.concatenate([dst, jnp.full((e_pad - e,), n, jnp.int32)])
    src_rows = src_pad.reshape(-1, CH)
    dst_rows = dst_pad.reshape(-1, CH)

    h = x
    for l in range(W.shape[0]):
        agg2 = _sc_agg(h, src_rows, dst_rows, n)
        h = _tc_update(h, agg2, W[l])
    return h
